# baseline (speedup 1.0000x reference)
import jax
import jax.numpy as jnp
from jax.experimental import pallas as pl
from jax.experimental.pallas import tpu as pltpu

C_IN = 12
C16 = 16
C_OUT = 32
K = 5
PAD = 2
LG = 4
HIDDEN = 32
OUT = 5
OUT_PAD = 128
TB = 2048


def _ecg_kernel(x_ref, w4_ref, wihft_ref, wihbt_ref, whhft_ref, whhbt_ref,
                bhhnt_ref, fcw_ref, fcb_ref, out_ref, f_scr):
    L, _, tb = x_ref.shape
    T = L // 2
    H = HIDDEN
    G6 = 6 * H
    w4 = w4_ref[...]

    zplane = jnp.zeros((C16, tb), jnp.bfloat16)
    ones8 = jnp.ones((8, tb), jnp.bfloat16)
    for g in range(L // LG):
        planes = [zplane if (l < 0 or l >= L) else x_ref[l]
                  for l in range(LG * g - PAD, LG * g + (K - PAD - 1) + LG)]
        bg = jnp.concatenate([planes[gi + k] for gi in range(LG)
                              for k in range(K)], axis=0)
        yg = jnp.maximum(jnp.dot(w4, bg,
                                 preferred_element_type=jnp.float32), 0.0)
        for p in range(LG // 2):
            f = jnp.maximum(yg[(2 * p) * C_OUT:(2 * p + 1) * C_OUT],
                            yg[(2 * p + 1) * C_OUT:(2 * p + 2) * C_OUT])
            f_scr[2 * g + p] = jnp.concatenate(
                [f.astype(jnp.bfloat16), ones8], axis=0)

    wihf = wihft_ref[...]
    wihb = wihbt_ref[...]
    whhf = whhft_ref[...]
    whhb = whhbt_ref[...]
    bhhn2 = jnp.concatenate([bhhnt_ref[...]] * (tb // 128), axis=1)
    bhf = bhhn2[:H]
    bhb = bhhn2[H:]

    def step(t, hs):
        hf, hb = hs
        gxf = jnp.dot(wihf, f_scr[t],
                      preferred_element_type=jnp.float32)
        gxb = jnp.dot(wihb, f_scr[T - 1 - t],
                      preferred_element_type=jnp.float32)
        ghf = jnp.dot(whhf, hf, preferred_element_type=jnp.float32)
        ghb = jnp.dot(whhb, hb, preferred_element_type=jnp.float32)
        rzf = jax.nn.sigmoid(gxf[:2 * H] + ghf[:2 * H])
        rzb = jax.nn.sigmoid(gxb[:2 * H] + ghb[:2 * H])
        nf = jnp.tanh(gxf[2 * H:] + rzf[:H] * (ghf[2 * H:] + bhf))
        nb = jnp.tanh(gxb[2 * H:] + rzb[:H] * (ghb[2 * H:] + bhb))
        hf = (1.0 - rzf[H:]) * nf + rzf[H:] * hf
        hb = (1.0 - rzb[H:]) * nb + rzb[H:] * hb
        return (hf, hb)

    h0 = jnp.zeros((H, tb), jnp.float32)
    hf, hb = jax.lax.fori_loop(0, T, step, (h0, h0), unroll=4)

    ht = jnp.transpose(jnp.concatenate([hf, hb], axis=0))
    out_ref[...] = jnp.dot(ht, fcw_ref[...],
                           preferred_element_type=jnp.float32) + fcb_ref[...]


def kernel(x, conv_w, conv_b, wih, bih, whh, bhh_n, fc_w, fc_b):
    B, C, L = x.shape
    assert C == C_IN and L % (2 * LG) == 0
    tb = TB if B % TB == 0 else B
    G = B // tb
    T = L // 2

    xe = jnp.concatenate(
        [x, jnp.ones((B, 1, L), jnp.float32),
         jnp.zeros((B, C16 - C_IN - 1, L), jnp.float32)], axis=1)
    xt = jnp.transpose(xe, (2, 1, 0)).astype(jnp.bfloat16)

    wk = jnp.transpose(conv_w.reshape(K, C_IN, C_OUT), (0, 2, 1))
    bias_col = jnp.zeros((K, C_OUT, 1), jnp.float32).at[PAD, :, 0].set(
        conv_b[0])
    wk16 = jnp.concatenate(
        [wk, bias_col, jnp.zeros((K, C_OUT, C16 - C_IN - 1), jnp.float32)],
        axis=2)
    wkc = jnp.transpose(wk16, (1, 0, 2)).reshape(C_OUT, K * C16)
    w4 = jnp.zeros((LG * C_OUT, LG * K * C16), jnp.float32)
    for gi in range(LG):
        w4 = w4.at[gi * C_OUT:(gi + 1) * C_OUT,
                   gi * K * C16:(gi + 1) * K * C16].set(wkc)
    w4 = w4.astype(jnp.bfloat16)

    H = HIDDEN
    wt = jnp.transpose(wih)
    bt = jnp.transpose(bih)
    z7 = jnp.zeros((3 * H, 7), jnp.float32)

    def _wih_dir(off):
        rows = [slice(off * H, (off + 1) * H),
                slice((2 + off) * H, (3 + off) * H),
                slice((4 + off) * H, (5 + off) * H)]
        w = jnp.concatenate([wt[s] for s in rows], axis=0)
        b = jnp.concatenate([bt[s] for s in rows], axis=0)
        return jnp.concatenate([w, b, z7], axis=1).astype(jnp.bfloat16)

    wihft = _wih_dir(0)
    wihbt = _wih_dir(1)
    whhft = jnp.concatenate(
        [jnp.transpose(whh[0:H, 0 * H:1 * H]),
         jnp.transpose(whh[0:H, 2 * H:3 * H]),
         jnp.transpose(whh[0:H, 4 * H:5 * H])], axis=0)
    whhbt = jnp.concatenate(
        [jnp.transpose(whh[H:2 * H, 1 * H:2 * H]),
         jnp.transpose(whh[H:2 * H, 3 * H:4 * H]),
         jnp.transpose(whh[H:2 * H, 5 * H:6 * H])], axis=0)
    bhhnt = jnp.broadcast_to(jnp.transpose(bhh_n), (2 * HIDDEN, 128))

    args = (xt, w4, wihft, wihbt, whhft, whhbt, bhhnt, fc_w, fc_b)

    def full_spec(a):
        return pl.BlockSpec(a.shape, lambda g, nd=a.ndim: (0,) * nd)

    in_specs = ([pl.BlockSpec((L, C16, tb), lambda g: (0, 0, g))]
                + [full_spec(a) for a in args[1:]])

    out = pl.pallas_call(
        _ecg_kernel,
        out_shape=jax.ShapeDtypeStruct((B, OUT_PAD), jnp.float32),
        grid_spec=pltpu.PrefetchScalarGridSpec(
            num_scalar_prefetch=0,
            grid=(G,),
            in_specs=in_specs,
            out_specs=pl.BlockSpec((tb, OUT_PAD), lambda g: (g, 0)),
            scratch_shapes=[pltpu.VMEM((T, 40, tb), jnp.bfloat16)],
        ),
        compiler_params=pltpu.CompilerParams(
            dimension_semantics=("parallel",),
            vmem_limit_bytes=100 * 1024 * 1024,
        ),
    )(*args)
    return out[:, :OUT]


# speedup vs baseline: 1.0546x; 1.0546x over previous
import jax
import jax.numpy as jnp
from jax.experimental import pallas as pl
from jax.experimental.pallas import tpu as pltpu

C_IN = 12
C16 = 16
C_OUT = 32
K = 5
PAD = 2
LG = 4
HIDDEN = 32
OUT = 5
OUT_PAD = 128
TB = 1024


def _ecg_kernel(x_ref, w4_ref, wihft_ref, wihbt_ref, whhft_ref, whhbt_ref,
                bhhnt_ref, fcw_ref, fcb_ref, out_ref, f_scr):
    L, _, tb = x_ref.shape
    T = L // 2
    H = HIDDEN
    G6 = 6 * H
    w4 = w4_ref[...]

    zplane = jnp.zeros((C16, tb), jnp.bfloat16)
    ones8 = jnp.ones((8, tb), jnp.bfloat16)
    for g in range(L // LG):
        planes = [zplane if (l < 0 or l >= L) else x_ref[l]
                  for l in range(LG * g - PAD, LG * g + (K - PAD - 1) + LG)]
        bg = jnp.concatenate([planes[gi + k] for gi in range(LG)
                              for k in range(K)], axis=0)
        yg = jnp.maximum(jnp.dot(w4, bg,
                                 preferred_element_type=jnp.float32), 0.0)
        for p in range(LG // 2):
            f = jnp.maximum(yg[(2 * p) * C_OUT:(2 * p + 1) * C_OUT],
                            yg[(2 * p + 1) * C_OUT:(2 * p + 2) * C_OUT])
            f_scr[2 * g + p] = jnp.concatenate(
                [f.astype(jnp.bfloat16), ones8], axis=0)

    wihf = wihft_ref[...]
    wihb = wihbt_ref[...]
    whhf = whhft_ref[...]
    whhb = whhbt_ref[...]
    bhhn2 = jnp.concatenate([bhhnt_ref[...]] * (tb // 128), axis=1)
    bhf = bhhn2[:H]
    bhb = bhhn2[H:]

    def step(t, hs):
        hf, hb = hs
        gxf = jnp.dot(wihf, f_scr[t],
                      preferred_element_type=jnp.float32)
        gxb = jnp.dot(wihb, f_scr[T - 1 - t],
                      preferred_element_type=jnp.float32)
        ghf = jnp.dot(whhf, hf, preferred_element_type=jnp.float32)
        ghb = jnp.dot(whhb, hb, preferred_element_type=jnp.float32)
        rzf = jax.nn.sigmoid(gxf[:2 * H] + ghf[:2 * H])
        rzb = jax.nn.sigmoid(gxb[:2 * H] + ghb[:2 * H])
        nf = jnp.tanh(gxf[2 * H:] + rzf[:H] * (ghf[2 * H:] + bhf))
        nb = jnp.tanh(gxb[2 * H:] + rzb[:H] * (ghb[2 * H:] + bhb))
        hf = (1.0 - rzf[H:]) * nf + rzf[H:] * hf
        hb = (1.0 - rzb[H:]) * nb + rzb[H:] * hb
        return (hf, hb)

    h0 = jnp.zeros((H, tb), jnp.float32)
    hf, hb = jax.lax.fori_loop(0, T, step, (h0, h0), unroll=4)

    ht = jnp.transpose(jnp.concatenate([hf, hb], axis=0))
    out_ref[...] = jnp.dot(ht, fcw_ref[...],
                           preferred_element_type=jnp.float32) + fcb_ref[...]


def kernel(x, conv_w, conv_b, wih, bih, whh, bhh_n, fc_w, fc_b):
    B, C, L = x.shape
    assert C == C_IN and L % (2 * LG) == 0
    tb = TB if B % TB == 0 else B
    G = B // tb
    T = L // 2

    xe = jnp.concatenate(
        [x, jnp.ones((B, 1, L), jnp.float32),
         jnp.zeros((B, C16 - C_IN - 1, L), jnp.float32)], axis=1)
    xt = jnp.transpose(xe, (2, 1, 0)).astype(jnp.bfloat16)

    wk = jnp.transpose(conv_w.reshape(K, C_IN, C_OUT), (0, 2, 1))
    bias_col = jnp.zeros((K, C_OUT, 1), jnp.float32).at[PAD, :, 0].set(
        conv_b[0])
    wk16 = jnp.concatenate(
        [wk, bias_col, jnp.zeros((K, C_OUT, C16 - C_IN - 1), jnp.float32)],
        axis=2)
    wkc = jnp.transpose(wk16, (1, 0, 2)).reshape(C_OUT, K * C16)
    w4 = jnp.zeros((LG * C_OUT, LG * K * C16), jnp.float32)
    for gi in range(LG):
        w4 = w4.at[gi * C_OUT:(gi + 1) * C_OUT,
                   gi * K * C16:(gi + 1) * K * C16].set(wkc)
    w4 = w4.astype(jnp.bfloat16)

    H = HIDDEN
    wt = jnp.transpose(wih)
    bt = jnp.transpose(bih)
    z7 = jnp.zeros((3 * H, 7), jnp.float32)

    def _wih_dir(off):
        rows = [slice(off * H, (off + 1) * H),
                slice((2 + off) * H, (3 + off) * H),
                slice((4 + off) * H, (5 + off) * H)]
        w = jnp.concatenate([wt[s] for s in rows], axis=0)
        b = jnp.concatenate([bt[s] for s in rows], axis=0)
        return jnp.concatenate([w, b, z7], axis=1).astype(jnp.bfloat16)

    wihft = _wih_dir(0)
    wihbt = _wih_dir(1)
    whhft = jnp.concatenate(
        [jnp.transpose(whh[0:H, 0 * H:1 * H]),
         jnp.transpose(whh[0:H, 2 * H:3 * H]),
         jnp.transpose(whh[0:H, 4 * H:5 * H])], axis=0)
    whhbt = jnp.concatenate(
        [jnp.transpose(whh[H:2 * H, 1 * H:2 * H]),
         jnp.transpose(whh[H:2 * H, 3 * H:4 * H]),
         jnp.transpose(whh[H:2 * H, 5 * H:6 * H])], axis=0)
    bhhnt = jnp.broadcast_to(jnp.transpose(bhh_n), (2 * HIDDEN, 128))

    args = (xt, w4, wihft, wihbt, whhft, whhbt, bhhnt, fc_w, fc_b)

    def full_spec(a):
        return pl.BlockSpec(a.shape, lambda g, nd=a.ndim: (0,) * nd)

    in_specs = ([pl.BlockSpec((L, C16, tb), lambda g: (0, 0, g))]
                + [full_spec(a) for a in args[1:]])

    out = pl.pallas_call(
        _ecg_kernel,
        out_shape=jax.ShapeDtypeStruct((B, OUT_PAD), jnp.float32),
        grid_spec=pltpu.PrefetchScalarGridSpec(
            num_scalar_prefetch=0,
            grid=(G,),
            in_specs=in_specs,
            out_specs=pl.BlockSpec((tb, OUT_PAD), lambda g: (g, 0)),
            scratch_shapes=[pltpu.VMEM((T, 40, tb), jnp.bfloat16)],
        ),
        compiler_params=pltpu.CompilerParams(
            dimension_semantics=("parallel",),
            vmem_limit_bytes=100 * 1024 * 1024,
        ),
    )(*args)
    return out[:, :OUT]


# speedup vs baseline: 1.0583x; 1.0035x over previous
import jax
import jax.numpy as jnp
from jax.experimental import pallas as pl
from jax.experimental.pallas import tpu as pltpu

C_IN = 12
C16 = 16
C_OUT = 32
K = 5
PAD = 2
LG = 4
HIDDEN = 32
OUT = 5
OUT_PAD = 128
TB = 1024


def _ecg_kernel(x_ref, w4_ref, wihft_ref, wihbt_ref, whhft_ref, whhbt_ref,
                bhhnt_ref, fcw_ref, fcb_ref, out_ref, f_scr):
    L, _, tb = x_ref.shape
    T = L // 2
    H = HIDDEN
    G6 = 6 * H
    w4 = w4_ref[...]

    zplane = jnp.zeros((C16, tb), jnp.bfloat16)
    ones8 = jnp.ones((8, tb), jnp.bfloat16)
    for g in range(L // LG):
        planes = [zplane if (l < 0 or l >= L) else x_ref[l]
                  for l in range(LG * g - PAD, LG * g + (K - PAD - 1) + LG)]
        bg = jnp.concatenate([planes[gi + k] for gi in range(LG)
                              for k in range(K)], axis=0)
        yg = jnp.maximum(jnp.dot(w4, bg,
                                 preferred_element_type=jnp.float32), 0.0)
        for p in range(LG // 2):
            f = jnp.maximum(yg[(2 * p) * C_OUT:(2 * p + 1) * C_OUT],
                            yg[(2 * p + 1) * C_OUT:(2 * p + 2) * C_OUT])
            f_scr[2 * g + p] = jnp.concatenate(
                [f.astype(jnp.bfloat16), ones8], axis=0)

    wihf = wihft_ref[...]
    wihb = wihbt_ref[...]
    whhf = whhft_ref[...]
    whhb = whhbt_ref[...]
    bhhn2 = jnp.concatenate([bhhnt_ref[...]] * (tb // 128), axis=1)
    bhf = bhhn2[:H]
    bhb = bhhn2[H:]

    def step(t, hs):
        hf, hb = hs
        gxf = jnp.dot(wihf, f_scr[t],
                      preferred_element_type=jnp.float32)
        gxb = jnp.dot(wihb, f_scr[T - 1 - t],
                      preferred_element_type=jnp.float32)
        ghf = jnp.dot(whhf, hf, preferred_element_type=jnp.float32)
        ghb = jnp.dot(whhb, hb, preferred_element_type=jnp.float32)
        rzf = jax.nn.sigmoid(gxf[:2 * H] + ghf[:2 * H])
        rzb = jax.nn.sigmoid(gxb[:2 * H] + ghb[:2 * H])
        nf = jnp.tanh(gxf[2 * H:] + rzf[:H] * (ghf[2 * H:] + bhf))
        nb = jnp.tanh(gxb[2 * H:] + rzb[:H] * (ghb[2 * H:] + bhb))
        hf = (1.0 - rzf[H:]) * nf + rzf[H:] * hf
        hb = (1.0 - rzb[H:]) * nb + rzb[H:] * hb
        return (hf, hb)

    h0 = jnp.zeros((H, tb), jnp.float32)
    hf, hb = jax.lax.fori_loop(0, T, step, (h0, h0), unroll=4)

    ht = jnp.transpose(jnp.concatenate([hf, hb], axis=0))
    out_ref[...] = jnp.dot(ht, fcw_ref[...],
                           preferred_element_type=jnp.float32) + fcb_ref[...]


def kernel(x, conv_w, conv_b, wih, bih, whh, bhh_n, fc_w, fc_b):
    B, C, L = x.shape
    assert C == C_IN and L % (2 * LG) == 0
    tb = TB if B % TB == 0 else B
    G = B // tb
    T = L // 2

    xe = jnp.concatenate(
        [x.astype(jnp.bfloat16), jnp.ones((B, 1, L), jnp.bfloat16),
         jnp.zeros((B, C16 - C_IN - 1, L), jnp.bfloat16)], axis=1)
    xt = jnp.transpose(xe, (2, 1, 0))

    wk = jnp.transpose(conv_w.reshape(K, C_IN, C_OUT), (0, 2, 1))
    bias_col = jnp.zeros((K, C_OUT, 1), jnp.float32).at[PAD, :, 0].set(
        conv_b[0])
    wk16 = jnp.concatenate(
        [wk, bias_col, jnp.zeros((K, C_OUT, C16 - C_IN - 1), jnp.float32)],
        axis=2)
    wkc = jnp.transpose(wk16, (1, 0, 2)).reshape(C_OUT, K * C16)
    w4 = jnp.zeros((LG * C_OUT, LG * K * C16), jnp.float32)
    for gi in range(LG):
        w4 = w4.at[gi * C_OUT:(gi + 1) * C_OUT,
                   gi * K * C16:(gi + 1) * K * C16].set(wkc)
    w4 = w4.astype(jnp.bfloat16)

    H = HIDDEN
    wt = jnp.transpose(wih)
    bt = jnp.transpose(bih)
    z7 = jnp.zeros((3 * H, 7), jnp.float32)

    def _wih_dir(off):
        rows = [slice(off * H, (off + 1) * H),
                slice((2 + off) * H, (3 + off) * H),
                slice((4 + off) * H, (5 + off) * H)]
        w = jnp.concatenate([wt[s] for s in rows], axis=0)
        b = jnp.concatenate([bt[s] for s in rows], axis=0)
        return jnp.concatenate([w, b, z7], axis=1).astype(jnp.bfloat16)

    wihft = _wih_dir(0)
    wihbt = _wih_dir(1)
    whhft = jnp.concatenate(
        [jnp.transpose(whh[0:H, 0 * H:1 * H]),
         jnp.transpose(whh[0:H, 2 * H:3 * H]),
         jnp.transpose(whh[0:H, 4 * H:5 * H])], axis=0)
    whhbt = jnp.concatenate(
        [jnp.transpose(whh[H:2 * H, 1 * H:2 * H]),
         jnp.transpose(whh[H:2 * H, 3 * H:4 * H]),
         jnp.transpose(whh[H:2 * H, 5 * H:6 * H])], axis=0)
    bhhnt = jnp.broadcast_to(jnp.transpose(bhh_n), (2 * HIDDEN, 128))

    args = (xt, w4, wihft, wihbt, whhft, whhbt, bhhnt, fc_w, fc_b)

    def full_spec(a):
        return pl.BlockSpec(a.shape, lambda g, nd=a.ndim: (0,) * nd)

    in_specs = ([pl.BlockSpec((L, C16, tb), lambda g: (0, 0, g))]
                + [full_spec(a) for a in args[1:]])

    out = pl.pallas_call(
        _ecg_kernel,
        out_shape=jax.ShapeDtypeStruct((B, OUT_PAD), jnp.float32),
        grid_spec=pltpu.PrefetchScalarGridSpec(
            num_scalar_prefetch=0,
            grid=(G,),
            in_specs=in_specs,
            out_specs=pl.BlockSpec((tb, OUT_PAD), lambda g: (g, 0)),
            scratch_shapes=[pltpu.VMEM((T, 40, tb), jnp.bfloat16)],
        ),
        compiler_params=pltpu.CompilerParams(
            dimension_semantics=("parallel",),
            vmem_limit_bytes=100 * 1024 * 1024,
        ),
    )(*args)
    return out[:, :OUT]


# speedup vs baseline: 1.1123x; 1.0510x over previous
import jax
import jax.numpy as jnp
from jax.experimental import pallas as pl
from jax.experimental.pallas import tpu as pltpu

C_IN = 12
C16 = 16
C_OUT = 32
K = 5
PAD = 2
LG = 4
HIDDEN = 32
OUT = 5
OUT_PAD = 128
TB = 1024


def _ecg_kernel(x_ref, w4_ref, wihft_ref, wihbt_ref, whhft_ref, whhbt_ref,
                bhhnt_ref, fcw_ref, fcb_ref, out_ref, f_scr):
    L, _, tb = x_ref.shape
    T = L // 2
    H = HIDDEN
    G6 = 6 * H
    w4 = w4_ref[...]

    zplane = jnp.zeros((C16, tb), jnp.bfloat16)
    ones8 = jnp.ones((8, tb), jnp.bfloat16)
    for g in range(L // LG):
        planes = [zplane if (l < 0 or l >= L) else x_ref[l]
                  for l in range(LG * g - PAD, LG * g + (K - PAD - 1) + LG)]
        bg = jnp.concatenate([planes[gi + k] for gi in range(LG)
                              for k in range(K)], axis=0)
        yg = jnp.maximum(jnp.dot(w4, bg,
                                 preferred_element_type=jnp.float32), 0.0)
        for p in range(LG // 2):
            f = jnp.maximum(yg[(2 * p) * C_OUT:(2 * p + 1) * C_OUT],
                            yg[(2 * p + 1) * C_OUT:(2 * p + 2) * C_OUT])
            f_scr[2 * g + p] = jnp.concatenate(
                [f.astype(jnp.bfloat16), ones8], axis=0)

    wihf = wihft_ref[...]
    wihb = wihbt_ref[...]
    whhf = whhft_ref[...]
    whhb = whhbt_ref[...]
    bhhn2 = jnp.concatenate([bhhnt_ref[...]] * (tb // 128), axis=1)
    bhf = bhhn2[:H]
    bhb = bhhn2[H:]

    def step(t, hs):
        hf, hb = hs
        gxf = jnp.dot(wihf, f_scr[t],
                      preferred_element_type=jnp.float32)
        gxb = jnp.dot(wihb, f_scr[T - 1 - t],
                      preferred_element_type=jnp.float32)
        ghf = jnp.dot(whhf, hf, preferred_element_type=jnp.float32)
        ghb = jnp.dot(whhb, hb, preferred_element_type=jnp.float32)
        rzf = jax.nn.sigmoid(gxf[:2 * H] + ghf[:2 * H])
        rzb = jax.nn.sigmoid(gxb[:2 * H] + ghb[:2 * H])
        nf = jnp.tanh(gxf[2 * H:] + rzf[:H] * (ghf[2 * H:] + bhf))
        nb = jnp.tanh(gxb[2 * H:] + rzb[:H] * (ghb[2 * H:] + bhb))
        hf = (1.0 - rzf[H:]) * nf + rzf[H:] * hf
        hb = (1.0 - rzb[H:]) * nb + rzb[H:] * hb
        return (hf, hb)

    h0 = jnp.zeros((H, tb), jnp.float32)
    hf, hb = jax.lax.fori_loop(0, T, step, (h0, h0), unroll=8)

    ht = jnp.transpose(jnp.concatenate([hf, hb], axis=0))
    out_ref[...] = jnp.dot(ht, fcw_ref[...],
                           preferred_element_type=jnp.float32) + fcb_ref[...]


def kernel(x, conv_w, conv_b, wih, bih, whh, bhh_n, fc_w, fc_b):
    B, C, L = x.shape
    assert C == C_IN and L % (2 * LG) == 0
    tb = TB if B % TB == 0 else B
    G = B // tb
    T = L // 2

    xe = jnp.concatenate(
        [x.astype(jnp.bfloat16), jnp.ones((B, 1, L), jnp.bfloat16),
         jnp.zeros((B, C16 - C_IN - 1, L), jnp.bfloat16)], axis=1)
    xt = jnp.transpose(xe, (2, 1, 0))

    wk = jnp.transpose(conv_w.reshape(K, C_IN, C_OUT), (0, 2, 1))
    bias_col = jnp.zeros((K, C_OUT, 1), jnp.float32).at[PAD, :, 0].set(
        conv_b[0])
    wk16 = jnp.concatenate(
        [wk, bias_col, jnp.zeros((K, C_OUT, C16 - C_IN - 1), jnp.float32)],
        axis=2)
    wkc = jnp.transpose(wk16, (1, 0, 2)).reshape(C_OUT, K * C16)
    w4 = jnp.zeros((LG * C_OUT, LG * K * C16), jnp.float32)
    for gi in range(LG):
        w4 = w4.at[gi * C_OUT:(gi + 1) * C_OUT,
                   gi * K * C16:(gi + 1) * K * C16].set(wkc)
    w4 = w4.astype(jnp.bfloat16)

    H = HIDDEN
    wt = jnp.transpose(wih)
    bt = jnp.transpose(bih)
    z7 = jnp.zeros((3 * H, 7), jnp.float32)

    def _wih_dir(off):
        rows = [slice(off * H, (off + 1) * H),
                slice((2 + off) * H, (3 + off) * H),
                slice((4 + off) * H, (5 + off) * H)]
        w = jnp.concatenate([wt[s] for s in rows], axis=0)
        b = jnp.concatenate([bt[s] for s in rows], axis=0)
        return jnp.concatenate([w, b, z7], axis=1).astype(jnp.bfloat16)

    wihft = _wih_dir(0)
    wihbt = _wih_dir(1)
    whhft = jnp.concatenate(
        [jnp.transpose(whh[0:H, 0 * H:1 * H]),
         jnp.transpose(whh[0:H, 2 * H:3 * H]),
         jnp.transpose(whh[0:H, 4 * H:5 * H])], axis=0)
    whhbt = jnp.concatenate(
        [jnp.transpose(whh[H:2 * H, 1 * H:2 * H]),
         jnp.transpose(whh[H:2 * H, 3 * H:4 * H]),
         jnp.transpose(whh[H:2 * H, 5 * H:6 * H])], axis=0)
    bhhnt = jnp.broadcast_to(jnp.transpose(bhh_n), (2 * HIDDEN, 128))

    args = (xt, w4, wihft, wihbt, whhft, whhbt, bhhnt, fc_w, fc_b)

    def full_spec(a):
        return pl.BlockSpec(a.shape, lambda g, nd=a.ndim: (0,) * nd)

    in_specs = ([pl.BlockSpec((L, C16, tb), lambda g: (0, 0, g))]
                + [full_spec(a) for a in args[1:]])

    out = pl.pallas_call(
        _ecg_kernel,
        out_shape=jax.ShapeDtypeStruct((B, OUT_PAD), jnp.float32),
        grid_spec=pltpu.PrefetchScalarGridSpec(
            num_scalar_prefetch=0,
            grid=(G,),
            in_specs=in_specs,
            out_specs=pl.BlockSpec((tb, OUT_PAD), lambda g: (g, 0)),
            scratch_shapes=[pltpu.VMEM((T, 40, tb), jnp.bfloat16)],
        ),
        compiler_params=pltpu.CompilerParams(
            dimension_semantics=("parallel",),
            vmem_limit_bytes=100 * 1024 * 1024,
        ),
    )(*args)
    return out[:, :OUT]


# speedup vs baseline: 1.1424x; 1.0271x over previous
import jax
import jax.numpy as jnp
from jax.experimental import pallas as pl
from jax.experimental.pallas import tpu as pltpu

C_IN = 12
C16 = 16
C_OUT = 32
K = 5
PAD = 2
LG = 4
HIDDEN = 32
OUT = 5
OUT_PAD = 128
TB = 1024


def _ecg_kernel(x_ref, w4_ref, wihft_ref, wihbt_ref, whhft_ref, whhbt_ref,
                bhhnt_ref, fcw_ref, fcb_ref, out_ref, f_scr):
    L, _, tb = x_ref.shape
    T = L // 2
    H = HIDDEN
    G6 = 6 * H
    w4 = w4_ref[...]

    zplane = jnp.zeros((C16, tb), jnp.bfloat16)
    ones8 = jnp.ones((8, tb), jnp.bfloat16)
    for g in range(L // LG):
        planes = [zplane if (l < 0 or l >= L) else x_ref[l]
                  for l in range(LG * g - PAD, LG * g + (K - PAD - 1) + LG)]
        bg = jnp.concatenate([planes[gi + k] for gi in range(LG)
                              for k in range(K)], axis=0)
        yg = jnp.maximum(jnp.dot(w4, bg,
                                 preferred_element_type=jnp.float32), 0.0)
        for p in range(LG // 2):
            f = jnp.maximum(yg[(2 * p) * C_OUT:(2 * p + 1) * C_OUT],
                            yg[(2 * p + 1) * C_OUT:(2 * p + 2) * C_OUT])
            f_scr[2 * g + p] = jnp.concatenate(
                [f.astype(jnp.bfloat16), ones8], axis=0)

    wihf = wihft_ref[...]
    wihb = wihbt_ref[...]
    whhf = whhft_ref[...]
    whhb = whhbt_ref[...]
    bhhn2 = jnp.concatenate([bhhnt_ref[...]] * (tb // 128), axis=1)
    bhf = bhhn2[:H]
    bhb = bhhn2[H:]

    def step(t, hs):
        hf, hb = hs
        gxf = jnp.dot(wihf, f_scr[t],
                      preferred_element_type=jnp.float32)
        gxb = jnp.dot(wihb, f_scr[T - 1 - t],
                      preferred_element_type=jnp.float32)
        ghf = jnp.dot(whhf, hf, preferred_element_type=jnp.float32)
        ghb = jnp.dot(whhb, hb, preferred_element_type=jnp.float32)
        rzf = jax.nn.sigmoid(gxf[:2 * H] + ghf[:2 * H])
        rzb = jax.nn.sigmoid(gxb[:2 * H] + ghb[:2 * H])
        nf = jnp.tanh(gxf[2 * H:] + rzf[:H] * (ghf[2 * H:] + bhf))
        nb = jnp.tanh(gxb[2 * H:] + rzb[:H] * (ghb[2 * H:] + bhb))
        hf = (1.0 - rzf[H:]) * nf + rzf[H:] * hf
        hb = (1.0 - rzb[H:]) * nb + rzb[H:] * hb
        return (hf, hb)

    h0 = jnp.zeros((H, tb), jnp.float32)
    hf, hb = jax.lax.fori_loop(0, T, step, (h0, h0), unroll=16)

    ht = jnp.transpose(jnp.concatenate([hf, hb], axis=0))
    out_ref[...] = jnp.dot(ht, fcw_ref[...],
                           preferred_element_type=jnp.float32) + fcb_ref[...]


def kernel(x, conv_w, conv_b, wih, bih, whh, bhh_n, fc_w, fc_b):
    B, C, L = x.shape
    assert C == C_IN and L % (2 * LG) == 0
    tb = TB if B % TB == 0 else B
    G = B // tb
    T = L // 2

    xe = jnp.concatenate(
        [x.astype(jnp.bfloat16), jnp.ones((B, 1, L), jnp.bfloat16),
         jnp.zeros((B, C16 - C_IN - 1, L), jnp.bfloat16)], axis=1)
    xt = jnp.transpose(xe, (2, 1, 0))

    wk = jnp.transpose(conv_w.reshape(K, C_IN, C_OUT), (0, 2, 1))
    bias_col = jnp.zeros((K, C_OUT, 1), jnp.float32).at[PAD, :, 0].set(
        conv_b[0])
    wk16 = jnp.concatenate(
        [wk, bias_col, jnp.zeros((K, C_OUT, C16 - C_IN - 1), jnp.float32)],
        axis=2)
    wkc = jnp.transpose(wk16, (1, 0, 2)).reshape(C_OUT, K * C16)
    w4 = jnp.zeros((LG * C_OUT, LG * K * C16), jnp.float32)
    for gi in range(LG):
        w4 = w4.at[gi * C_OUT:(gi + 1) * C_OUT,
                   gi * K * C16:(gi + 1) * K * C16].set(wkc)
    w4 = w4.astype(jnp.bfloat16)

    H = HIDDEN
    wt = jnp.transpose(wih)
    bt = jnp.transpose(bih)
    z7 = jnp.zeros((3 * H, 7), jnp.float32)

    def _wih_dir(off):
        rows = [slice(off * H, (off + 1) * H),
                slice((2 + off) * H, (3 + off) * H),
                slice((4 + off) * H, (5 + off) * H)]
        w = jnp.concatenate([wt[s] for s in rows], axis=0)
        b = jnp.concatenate([bt[s] for s in rows], axis=0)
        return jnp.concatenate([w, b, z7], axis=1).astype(jnp.bfloat16)

    wihft = _wih_dir(0)
    wihbt = _wih_dir(1)
    whhft = jnp.concatenate(
        [jnp.transpose(whh[0:H, 0 * H:1 * H]),
         jnp.transpose(whh[0:H, 2 * H:3 * H]),
         jnp.transpose(whh[0:H, 4 * H:5 * H])], axis=0)
    whhbt = jnp.concatenate(
        [jnp.transpose(whh[H:2 * H, 1 * H:2 * H]),
         jnp.transpose(whh[H:2 * H, 3 * H:4 * H]),
         jnp.transpose(whh[H:2 * H, 5 * H:6 * H])], axis=0)
    bhhnt = jnp.broadcast_to(jnp.transpose(bhh_n), (2 * HIDDEN, 128))

    args = (xt, w4, wihft, wihbt, whhft, whhbt, bhhnt, fc_w, fc_b)

    def full_spec(a):
        return pl.BlockSpec(a.shape, lambda g, nd=a.ndim: (0,) * nd)

    in_specs = ([pl.BlockSpec((L, C16, tb), lambda g: (0, 0, g))]
                + [full_spec(a) for a in args[1:]])

    out = pl.pallas_call(
        _ecg_kernel,
        out_shape=jax.ShapeDtypeStruct((B, OUT_PAD), jnp.float32),
        grid_spec=pltpu.PrefetchScalarGridSpec(
            num_scalar_prefetch=0,
            grid=(G,),
            in_specs=in_specs,
            out_specs=pl.BlockSpec((tb, OUT_PAD), lambda g: (g, 0)),
            scratch_shapes=[pltpu.VMEM((T, 40, tb), jnp.bfloat16)],
        ),
        compiler_params=pltpu.CompilerParams(
            dimension_semantics=("parallel",),
            vmem_limit_bytes=100 * 1024 * 1024,
        ),
    )(*args)
    return out[:, :OUT]


# speedup vs baseline: 1.1555x; 1.0114x over previous
import jax
import jax.numpy as jnp
from jax.experimental import pallas as pl
from jax.experimental.pallas import tpu as pltpu

C_IN = 12
C16 = 16
C_OUT = 32
K = 5
PAD = 2
LG = 4
HIDDEN = 32
OUT = 5
OUT_PAD = 128
TB = 1024


def _ecg_kernel(x_ref, w4_ref, wihft_ref, wihbt_ref, whhft_ref, whhbt_ref,
                bhhnt_ref, fcw_ref, fcb_ref, out_ref, f_scr):
    L, _, tb = x_ref.shape
    T = L // 2
    H = HIDDEN
    G6 = 6 * H
    w4 = w4_ref[...]

    zplane = jnp.zeros((C16, tb), jnp.bfloat16)
    ones8 = jnp.ones((8, tb), jnp.bfloat16)
    for g in range(L // LG):
        planes = [zplane if (l < 0 or l >= L) else x_ref[l]
                  for l in range(LG * g - PAD, LG * g + (K - PAD - 1) + LG)]
        bg = jnp.concatenate([planes[gi + k] for gi in range(LG)
                              for k in range(K)], axis=0)
        yg = jnp.maximum(jnp.dot(w4, bg,
                                 preferred_element_type=jnp.float32), 0.0)
        for p in range(LG // 2):
            f = jnp.maximum(yg[(2 * p) * C_OUT:(2 * p + 1) * C_OUT],
                            yg[(2 * p + 1) * C_OUT:(2 * p + 2) * C_OUT])
            f_scr[2 * g + p] = jnp.concatenate(
                [f.astype(jnp.bfloat16), ones8], axis=0)

    wihf = wihft_ref[...]
    wihb = wihbt_ref[...]
    whhf = whhft_ref[...]
    whhb = whhbt_ref[...]
    bhhn2 = jnp.concatenate([bhhnt_ref[...]] * (tb // 128), axis=1)
    bhf = bhhn2[:H]
    bhb = bhhn2[H:]

    def step(t, hs):
        hf, hb = hs
        gxf = jnp.dot(wihf, f_scr[t],
                      preferred_element_type=jnp.float32)
        gxb = jnp.dot(wihb, f_scr[T - 1 - t],
                      preferred_element_type=jnp.float32)
        ghf = jnp.dot(whhf, hf, preferred_element_type=jnp.float32)
        ghb = jnp.dot(whhb, hb, preferred_element_type=jnp.float32)
        rzf = jax.nn.sigmoid(gxf[:2 * H] + ghf[:2 * H])
        rzb = jax.nn.sigmoid(gxb[:2 * H] + ghb[:2 * H])
        nf = jnp.tanh(gxf[2 * H:] + rzf[:H] * (ghf[2 * H:] + bhf))
        nb = jnp.tanh(gxb[2 * H:] + rzb[:H] * (ghb[2 * H:] + bhb))
        hf = (1.0 - rzf[H:]) * nf + rzf[H:] * hf
        hb = (1.0 - rzb[H:]) * nb + rzb[H:] * hb
        return (hf, hb)

    h0 = jnp.zeros((H, tb), jnp.float32)
    hf, hb = jax.lax.fori_loop(0, T, step, (h0, h0), unroll=32)

    ht = jnp.transpose(jnp.concatenate([hf, hb], axis=0))
    out_ref[...] = jnp.dot(ht, fcw_ref[...],
                           preferred_element_type=jnp.float32) + fcb_ref[...]


def kernel(x, conv_w, conv_b, wih, bih, whh, bhh_n, fc_w, fc_b):
    B, C, L = x.shape
    assert C == C_IN and L % (2 * LG) == 0
    tb = TB if B % TB == 0 else B
    G = B // tb
    T = L // 2

    xe = jnp.concatenate(
        [x.astype(jnp.bfloat16), jnp.ones((B, 1, L), jnp.bfloat16),
         jnp.zeros((B, C16 - C_IN - 1, L), jnp.bfloat16)], axis=1)
    xt = jnp.transpose(xe, (2, 1, 0))

    wk = jnp.transpose(conv_w.reshape(K, C_IN, C_OUT), (0, 2, 1))
    bias_col = jnp.zeros((K, C_OUT, 1), jnp.float32).at[PAD, :, 0].set(
        conv_b[0])
    wk16 = jnp.concatenate(
        [wk, bias_col, jnp.zeros((K, C_OUT, C16 - C_IN - 1), jnp.float32)],
        axis=2)
    wkc = jnp.transpose(wk16, (1, 0, 2)).reshape(C_OUT, K * C16)
    w4 = jnp.zeros((LG * C_OUT, LG * K * C16), jnp.float32)
    for gi in range(LG):
        w4 = w4.at[gi * C_OUT:(gi + 1) * C_OUT,
                   gi * K * C16:(gi + 1) * K * C16].set(wkc)
    w4 = w4.astype(jnp.bfloat16)

    H = HIDDEN
    wt = jnp.transpose(wih)
    bt = jnp.transpose(bih)
    z7 = jnp.zeros((3 * H, 7), jnp.float32)

    def _wih_dir(off):
        rows = [slice(off * H, (off + 1) * H),
                slice((2 + off) * H, (3 + off) * H),
                slice((4 + off) * H, (5 + off) * H)]
        w = jnp.concatenate([wt[s] for s in rows], axis=0)
        b = jnp.concatenate([bt[s] for s in rows], axis=0)
        return jnp.concatenate([w, b, z7], axis=1).astype(jnp.bfloat16)

    wihft = _wih_dir(0)
    wihbt = _wih_dir(1)
    whhft = jnp.concatenate(
        [jnp.transpose(whh[0:H, 0 * H:1 * H]),
         jnp.transpose(whh[0:H, 2 * H:3 * H]),
         jnp.transpose(whh[0:H, 4 * H:5 * H])], axis=0)
    whhbt = jnp.concatenate(
        [jnp.transpose(whh[H:2 * H, 1 * H:2 * H]),
         jnp.transpose(whh[H:2 * H, 3 * H:4 * H]),
         jnp.transpose(whh[H:2 * H, 5 * H:6 * H])], axis=0)
    bhhnt = jnp.broadcast_to(jnp.transpose(bhh_n), (2 * HIDDEN, 128))

    args = (xt, w4, wihft, wihbt, whhft, whhbt, bhhnt, fc_w, fc_b)

    def full_spec(a):
        return pl.BlockSpec(a.shape, lambda g, nd=a.ndim: (0,) * nd)

    in_specs = ([pl.BlockSpec((L, C16, tb), lambda g: (0, 0, g))]
                + [full_spec(a) for a in args[1:]])

    out = pl.pallas_call(
        _ecg_kernel,
        out_shape=jax.ShapeDtypeStruct((B, OUT_PAD), jnp.float32),
        grid_spec=pltpu.PrefetchScalarGridSpec(
            num_scalar_prefetch=0,
            grid=(G,),
            in_specs=in_specs,
            out_specs=pl.BlockSpec((tb, OUT_PAD), lambda g: (g, 0)),
            scratch_shapes=[pltpu.VMEM((T, 40, tb), jnp.bfloat16)],
        ),
        compiler_params=pltpu.CompilerParams(
            dimension_semantics=("parallel",),
            vmem_limit_bytes=100 * 1024 * 1024,
        ),
    )(*args)
    return out[:, :OUT]
